# Initial kernel scaffold; baseline (speedup 1.0000x reference)
import jax
import jax.numpy as jnp
from jax.experimental import pallas as pl


def kernel(x, conv0_w, conv0_b, conv0_gamma, conv0_beta, conv0_mean, conv0_var, conv1_w, conv1_b, conv1_gamma, conv1_beta, conv1_mean, conv1_var, conv2_w, conv2_b, conv2_gamma, conv2_beta, conv2_mean, conv2_var, conv3_w, conv3_b, conv3_gamma, conv3_beta, conv3_mean, conv3_var, conv4_w, conv4_b, conv4_gamma, conv4_beta, conv4_mean, conv4_var, fc0_w, fc0_b, fc1_w, fc1_b, fc2_w, fc2_b):
    raise NotImplementedError("write your pallas kernel here")


# speedup vs baseline: 3.0951x; 3.0951x over previous
import functools

import jax
import jax.numpy as jnp
from jax.experimental import pallas as pl
from jax.experimental.pallas import tpu as pltpu

BF = jnp.bfloat16


def _rup(a, m):
    return (a + m - 1) // m * m


def _mm_pool_kernel(p_ref, w_ref, scale_ref, shift_ref, o_ref, *, HC, W, C):
    y = jnp.dot(p_ref[0], w_ref[...], preferred_element_type=jnp.float32)
    y = y * scale_ref[...] + shift_ref[...]
    y = jnp.maximum(y, 0.0).astype(BF)
    y = y.reshape(HC // 2, 2, W, C)
    y = jnp.maximum(y[:, 0], y[:, 1])
    o_ref[...] = y.reshape(1, HC // 2, W, C)


def _conv_pool_kernel(x_ref, w_ref, scale_ref, shift_ref, o_ref,
                      pad_scr, p_scr, *, G, H, W, Cin, Cout, pad, final):
    Hf, Wf = H + 2 * pad, W + 2 * pad
    Ho, Wo = Hf - 2, Wf - 2
    Wr = _rup(Wo, 8)
    He, We = Ho - Ho % 2, Wo - Wo % 2
    Hp = He // 2

    x = x_ref[...]
    x = jnp.maximum(x[..., :Cin], x[..., Cin:])

    if pad:
        pad_scr[:, 0:pad, :, :] = jnp.zeros((G, pad, Wf, Cin), BF)
        pad_scr[:, Hf - pad:Hf, :, :] = jnp.zeros((G, pad, Wf, Cin), BF)
        pad_scr[:, :, 0:pad, :] = jnp.zeros((G, Hf, pad, Cin), BF)
        pad_scr[:, :, Wf - pad:Wf, :] = jnp.zeros((G, Hf, pad, Cin), BF)
        pad_scr[:, pad:pad + H, pad:pad + W, :] = x
    else:
        pad_scr[...] = x

    for t in range(9):
        di, dj = t // 3, t % 3
        xs = pad_scr[:, di:di + Ho, dj:dj + Wo, :]
        p_scr[:, 0:Wo, t * Cin:(t + 1) * Cin] = xs.reshape(G * Ho, Wo, Cin)

    y = jnp.dot(p_scr[...].reshape(G * Ho * Wr, 9 * Cin), w_ref[...],
                preferred_element_type=jnp.float32)
    y = y * scale_ref[...] + shift_ref[...]
    y = jnp.maximum(y, 0.0).astype(BF)

    y = y.reshape(G, Ho, Wr, Cout)[:, :He]
    y = y.reshape(G * Hp, 2, Wr, Cout)
    y = jnp.maximum(y[:, 0], y[:, 1])[:, :We, :]

    if final:
        parts = [jnp.maximum(y[:, 2 * w:2 * w + 1, :],
                             y[:, 2 * w + 1:2 * w + 2, :])
                 for w in range(We // 2)]
        y = jnp.concatenate(parts, axis=1)
        o_ref[...] = y.reshape(G, Hp, We // 2, Cout)
    else:
        o_ref[...] = y.reshape(G, Hp, We, Cout)


def _conv_pool(x, w_mat, scale, shift, *, G, H, W, Cin, Cout, pad, final):
    B = x.shape[0]
    Hf, Wf = H + 2 * pad, W + 2 * pad
    Ho, Wo = Hf - 2, Wf - 2
    Wr = _rup(Wo, 8)
    He, We = Ho - Ho % 2, Wo - Wo % 2
    Hp = He // 2
    Wout = We // 2 if final else We

    return pl.pallas_call(
        functools.partial(_conv_pool_kernel, G=G, H=H, W=W, Cin=Cin,
                          Cout=Cout, pad=pad, final=final),
        out_shape=jax.ShapeDtypeStruct((B, Hp, Wout, Cout), BF),
        grid_spec=pltpu.PrefetchScalarGridSpec(
            num_scalar_prefetch=0,
            grid=(B // G,),
            in_specs=[
                pl.BlockSpec((G,) + x.shape[1:], lambda i: (i, 0, 0, 0)),
                pl.BlockSpec(w_mat.shape, lambda i: (0, 0)),
                pl.BlockSpec((1, Cout), lambda i: (0, 0)),
                pl.BlockSpec((1, Cout), lambda i: (0, 0)),
            ],
            out_specs=pl.BlockSpec((G, Hp, Wout, Cout),
                                   lambda i: (i, 0, 0, 0)),
            scratch_shapes=[
                pltpu.VMEM((G, Hf, Wf, Cin), BF),
                pltpu.VMEM((G * Ho, Wr, 9 * Cin), BF),
            ],
        ),
        compiler_params=pltpu.CompilerParams(
            dimension_semantics=("parallel",)),
    )(x, w_mat, scale, shift)


def _fc_kernel(f_ref, w0_ref, b0_ref, w1_ref, b1_ref, w2_ref, b2_ref, o_ref):
    h = jnp.dot(f_ref[...], w0_ref[...], preferred_element_type=jnp.float32)
    h = jnp.maximum(h + b0_ref[...], 0.0).astype(BF)
    h = jnp.dot(h, w1_ref[...], preferred_element_type=jnp.float32)
    h = jnp.maximum(h + b1_ref[...], 0.0).astype(BF)
    y = jnp.dot(h, w2_ref[...], preferred_element_type=jnp.float32)
    o_ref[...] = y + b2_ref[...]


def _affine(gamma, beta, mean, var, conv_b, eps=1e-5):
    scale = gamma / jnp.sqrt(var + eps)
    shift = (conv_b - mean) * scale + beta
    return (scale.reshape(1, -1).astype(jnp.float32),
            shift.reshape(1, -1).astype(jnp.float32))


def kernel(x, conv0_w, conv0_b, conv0_gamma, conv0_beta, conv0_mean, conv0_var, conv1_w, conv1_b, conv1_gamma, conv1_beta, conv1_mean, conv1_var, conv2_w, conv2_b, conv2_gamma, conv2_beta, conv2_mean, conv2_var, conv3_w, conv3_b, conv3_gamma, conv3_beta, conv3_mean, conv3_var, conv4_w, conv4_b, conv4_gamma, conv4_beta, conv4_mean, conv4_var, fc0_w, fc0_b, fc1_w, fc1_b, fc2_w, fc2_b):
    B = x.shape[0]

    xb = jnp.transpose(x, (0, 2, 3, 1)).astype(BF)
    xp = jnp.pad(xb, ((0, 0), (1, 1), (1, 1), (0, 0)))
    cols = [xp[:, di:di + 128, dj:dj + 128, :]
            for di in range(3) for dj in range(3)]
    patches = jnp.concatenate(cols, -1).reshape(B, 128 * 128, 27)
    w0m = conv0_w.reshape(27, 64).astype(BF)
    s0, t0 = _affine(conv0_gamma, conv0_beta, conv0_mean, conv0_var, conv0_b)

    a = pl.pallas_call(
        functools.partial(_mm_pool_kernel, HC=32, W=128, C=64),
        out_shape=jax.ShapeDtypeStruct((B, 64, 128, 64), BF),
        grid_spec=pltpu.PrefetchScalarGridSpec(
            num_scalar_prefetch=0,
            grid=(B, 4),
            in_specs=[
                pl.BlockSpec((1, 4096, 27), lambda i, j: (i, j, 0)),
                pl.BlockSpec((27, 64), lambda i, j: (0, 0)),
                pl.BlockSpec((1, 64), lambda i, j: (0, 0)),
                pl.BlockSpec((1, 64), lambda i, j: (0, 0)),
            ],
            out_specs=pl.BlockSpec((1, 16, 128, 64),
                                   lambda i, j: (i, j, 0, 0)),
        ),
        compiler_params=pltpu.CompilerParams(
            dimension_semantics=("parallel", "parallel")),
    )(patches, w0m, s0, t0)

    convs = [
        (conv1_w, conv1_b, conv1_gamma, conv1_beta, conv1_mean, conv1_var,
         1, 1, 64, 64, False),
        (conv2_w, conv2_b, conv2_gamma, conv2_beta, conv2_mean, conv2_var,
         0, 1, 32, 32, False),
        (conv3_w, conv3_b, conv3_gamma, conv3_beta, conv3_mean, conv3_var,
         1, 4, 15, 15, False),
        (conv4_w, conv4_b, conv4_gamma, conv4_beta, conv4_mean, conv4_var,
         1, 8, 7, 7, True),
    ]
    for w, b, gamma, beta, mean, var, pad, G, H, W, final in convs:
        Cin, Cout = w.shape[2], w.shape[3]
        a = a.reshape(B, a.shape[1], a.shape[2] // 2, 2 * a.shape[3])
        wm = w.reshape(9 * Cin, Cout).astype(BF)
        sc, sh = _affine(gamma, beta, mean, var, b)
        a = _conv_pool(a, wm, sc, sh, G=G, H=H, W=W, Cin=Cin, Cout=Cout,
                       pad=pad, final=final)

    feat = jnp.transpose(a, (0, 3, 1, 2)).reshape(B, -1).astype(BF)
    out = pl.pallas_call(
        _fc_kernel,
        out_shape=jax.ShapeDtypeStruct((B, 10), jnp.float32),
    )(feat, fc0_w.astype(BF), fc0_b.reshape(1, -1),
      fc1_w.astype(BF), fc1_b.reshape(1, -1),
      fc2_w.astype(BF), fc2_b.reshape(1, -1))
    return out
